# baseline (speedup 1.0000x reference)
import functools

import jax
import jax.numpy as jnp
from jax import lax
from jax.experimental import pallas as pl
from jax.experimental.pallas import tpu as pltpu
from jax.experimental.pallas import tpu_sc as plsc

_B = 16384
_D = 128
_NC = 2
_NS = 16
_NW = _NC * _NS
_BPW = _B // _NW
_CH = _BPW // 128
_EPS = 1e-5

_BLK = 1024
_NBLK = _B // _BLK


def _sc_gather_body(uidx_hbm, midx_hbm, uemb_hbm, memb_hbm, ubias_hbm,
                    mbias_hbm, ue_out, me_out, ub_out, mb_out,
                    idx_u, idx_m, rows, brows_u, brows_m, sem_e, sem_b):
    wid = lax.axis_index("s") * _NC + lax.axis_index("c")
    base = wid * _BPW
    pltpu.sync_copy(uidx_hbm.at[wid], idx_u)
    pltpu.sync_copy(midx_hbm.at[wid], idx_m)
    hb = []
    for j in range(_CH):
        hb.append(pltpu.async_copy(ubias_hbm.at[idx_u.at[j]],
                                   brows_u.at[pl.ds(j * 128, 128)], sem_b))
        hb.append(pltpu.async_copy(mbias_hbm.at[idx_m.at[j]],
                                   brows_m.at[pl.ds(j * 128, 128)], sem_b))
    he = [pltpu.async_copy(uemb_hbm.at[idx_u.at[j]],
                           rows.at[pl.ds(j * 128, 128)], sem_e)
          for j in range(_CH)]
    for h in he:
        h.wait()
    pltpu.sync_copy(rows, ue_out.at[pl.ds(base, _BPW)])
    he = [pltpu.async_copy(memb_hbm.at[idx_m.at[j]],
                           rows.at[pl.ds(j * 128, 128)], sem_e)
          for j in range(_CH)]
    for h in hb:
        h.wait()
    pltpu.sync_copy(brows_u, ub_out.at[pl.ds(base, _BPW)])
    pltpu.sync_copy(brows_m, mb_out.at[pl.ds(base, _BPW)])
    for h in he:
        h.wait()
    pltpu.sync_copy(rows, me_out.at[pl.ds(base, _BPW)])


def _sc_gather(uidx, midx, uemb, memb, ubias, mbias):
    mesh = plsc.VectorSubcoreMesh(core_axis_name="c", subcore_axis_name="s")
    fn = pl.kernel(
        _sc_gather_body,
        mesh=mesh,
        out_type=(
            jax.ShapeDtypeStruct((_B, _D), jnp.float32),
            jax.ShapeDtypeStruct((_B, _D), jnp.float32),
            jax.ShapeDtypeStruct((_B,), jnp.float32),
            jax.ShapeDtypeStruct((_B,), jnp.float32),
        ),
        scratch_types=[
            pltpu.VMEM((_CH, 128), jnp.int32),
            pltpu.VMEM((_CH, 128), jnp.int32),
            pltpu.VMEM((_BPW, _D), jnp.float32),
            pltpu.VMEM((_BPW,), jnp.float32),
            pltpu.VMEM((_BPW,), jnp.float32),
            pltpu.SemaphoreType.DMA,
            pltpu.SemaphoreType.DMA,
        ],
    )
    return fn(uidx, midx, uemb, memb, ubias, mbias)


def _affine(s_sum, s_sq, g, be):
    mu = s_sum * (1.0 / _B)
    var = s_sq * (1.0 / _B) - mu * mu
    a = g * lax.rsqrt(var + _EPS)
    return a, be - mu * a


def _colstats(y, ones_row):
    s = jnp.dot(ones_row, y, preferred_element_type=jnp.float32)
    sq = jnp.dot(ones_row, y * y, preferred_element_type=jnp.float32)
    return jnp.concatenate([s, sq], axis=0)


def _acc(ref, i, val):
    @pl.when(i == 0)
    def _():
        ref[...] = val

    @pl.when(i > 0)
    def _():
        ref[...] += val


def _fused_body(ue_ref, me_ref, gu_ref, bu_ref, gm_ref, bm_ref,
                w1t_ref, b1_ref, wrt_ref, sc_ref, ub_ref, mb_ref,
                g1_ref, be1_ref, w2t_ref, b2_ref,
                g2_ref, be2_ref, w3t_ref, b3_ref,
                g3_ref, be3_ref, wft_ref, bf_ref,
                out_ref,
                ue_s, me_s, y_s,
                s0_s, s1_s, s2_s, s3_s):
    p = pl.program_id(0)
    i = pl.program_id(1)
    rows = pl.ds(i * _BLK, _BLK)
    ones_row = jnp.ones((1, _BLK), jnp.float32)

    def _comb():
        s0 = s0_s[...]
        au, cu = _affine(s0[0:1], s0[1:2], gu_ref[...], bu_ref[...])
        am, cm = _affine(s0[2:3], s0[3:4], gm_ref[...], bm_ref[...])
        return jnp.concatenate([ue_s[rows, :] * au + cu,
                                me_s[rows, :] * am + cm], axis=1)

    @pl.when(p == 0)
    def _phase0():
        ue = ue_ref[...]
        me = me_ref[...]
        ue_s[rows, :] = ue
        me_s[rows, :] = me
        st = jnp.concatenate([_colstats(ue, ones_row),
                              _colstats(me, ones_row)], axis=0)
        _acc(s0_s, i, st)

    @pl.when(p == 1)
    def _phase1():
        y1 = jnp.dot(_comb(), w1t_ref[...],
                     preferred_element_type=jnp.float32) + b1_ref[...]
        y_s[rows, :] = y1
        _acc(s1_s, i, _colstats(y1, ones_row))

    @pl.when(p == 2)
    def _phase2():
        s1 = s1_s[...]
        a, c = _affine(s1[0:1], s1[1:2], g1_ref[...], be1_ref[...])
        x = jnp.maximum(y_s[rows, :] * a + c, 0.0)
        y2 = jnp.dot(x, w2t_ref[...],
                     preferred_element_type=jnp.float32) + b2_ref[...]
        y_s[rows, 0:256] = y2
        _acc(s2_s, i, _colstats(y2, ones_row))

    @pl.when(p == 3)
    def _phase3():
        s2 = s2_s[...]
        a, c = _affine(s2[0:1], s2[1:2], g2_ref[...], be2_ref[...])
        x = jnp.maximum(y_s[rows, 0:256] * a + c, 0.0)
        y3 = jnp.dot(x, w3t_ref[...],
                     preferred_element_type=jnp.float32) + b3_ref[...]
        y_s[rows, 0:128] = y3
        _acc(s3_s, i, _colstats(y3, ones_row))

    @pl.when(p == 4)
    def _phase4():
        s3 = s3_s[...]
        a, c = _affine(s3[0:1], s3[1:2], g3_ref[...], be3_ref[...])
        x = jnp.maximum(y_s[rows, 0:128] * a + c, 0.0)
        res = jnp.dot(_comb(), wrt_ref[...],
                      preferred_element_type=jnp.float32)
        out_ref[...] = (jnp.dot(x, wft_ref[...],
                                preferred_element_type=jnp.float32)
                        + res + sc_ref[0, 0] + bf_ref[0, 0]
                        + ub_ref[...] + mb_ref[...])


def _emb_spec():
    return pl.BlockSpec((_BLK, _D), lambda p, i: (jnp.where(p == 0, i, 0), 0))


def _bias_spec():
    return pl.BlockSpec((_BLK, 1), lambda p, i: (jnp.where(p == 4, i, 0), 0))


def _full_spec(shape):
    nd = len(shape)
    return pl.BlockSpec(shape, lambda p, i: (0,) * nd)


def kernel(users, movies, user_emb, movie_emb, user_bias_t, movie_bias_t,
           global_bias, gu, bu, gm, bm, W1, b1, g1, be1, W2, b2, g2, be2,
           W3, b3, g3, be3, Wf, bf, Wr, br):
    uidx = users.astype(jnp.int32).reshape(_NW, _CH, 128)
    midx = movies.astype(jnp.int32).reshape(_NW, _CH, 128)
    ue, me, ubg, mbg = _sc_gather(uidx, midx, user_emb, movie_emb,
                                  user_bias_t.reshape(-1),
                                  movie_bias_t.reshape(-1))

    f32 = jnp.float32
    scalar_c = (global_bias + br).reshape(1, 1)
    out = pl.pallas_call(
        _fused_body,
        grid=(5, _NBLK),
        in_specs=[_emb_spec(), _emb_spec(),
                  _full_spec((1, _D)), _full_spec((1, _D)),
                  _full_spec((1, _D)), _full_spec((1, _D)),
                  _full_spec((2 * _D, 512)), _full_spec((1, 512)),
                  _full_spec((2 * _D, 1)), _full_spec((1, 1)),
                  _bias_spec(), _bias_spec(),
                  _full_spec((1, 512)), _full_spec((1, 512)),
                  _full_spec((512, 256)), _full_spec((1, 256)),
                  _full_spec((1, 256)), _full_spec((1, 256)),
                  _full_spec((256, _D)), _full_spec((1, _D)),
                  _full_spec((1, _D)), _full_spec((1, _D)),
                  _full_spec((_D, 1)), _full_spec((1, 1))],
        out_specs=pl.BlockSpec((_BLK, 1),
                               lambda p, i: (jnp.where(p == 4, i, 0), 0)),
        out_shape=jax.ShapeDtypeStruct((_B, 1), f32),
        scratch_shapes=[
            pltpu.VMEM((_B, _D), f32),
            pltpu.VMEM((_B, _D), f32),
            pltpu.VMEM((_B, 512), f32),
            pltpu.VMEM((4, _D), f32),
            pltpu.VMEM((2, 512), f32),
            pltpu.VMEM((2, 256), f32),
            pltpu.VMEM((2, _D), f32),
        ],
        compiler_params=pltpu.CompilerParams(
            vmem_limit_bytes=110 * 1024 * 1024,
        ),
    )(ue, me, gu.reshape(1, -1), bu.reshape(1, -1),
      gm.reshape(1, -1), bm.reshape(1, -1), W1.T, b1.reshape(1, -1),
      Wr.T, scalar_c, ubg.reshape(-1, 1), mbg.reshape(-1, 1),
      g1.reshape(1, -1), be1.reshape(1, -1), W2.T, b2.reshape(1, -1),
      g2.reshape(1, -1), be2.reshape(1, -1), W3.T, b3.reshape(1, -1),
      g3.reshape(1, -1), be3.reshape(1, -1), Wf.T, bf.reshape(1, 1))

    return out.reshape(_B)


# speedup vs baseline: 1.0032x; 1.0032x over previous
import functools

import jax
import jax.numpy as jnp
from jax import lax
from jax.experimental import pallas as pl
from jax.experimental.pallas import tpu as pltpu
from jax.experimental.pallas import tpu_sc as plsc

_B = 16384
_D = 128
_NC = 2
_NS = 16
_NW = _NC * _NS
_BPW = _B // _NW
_CH = _BPW // 128
_EPS = 1e-5

_BLK = 1024
_NBLK = _B // _BLK


def _sc_gather_body(uidx_hbm, midx_hbm, uemb_hbm, memb_hbm, ubias_hbm,
                    mbias_hbm, ue_out, me_out, ub_out, mb_out,
                    idx_u, idx_m, rows, brows_u, brows_m,
                    sem_g, sem_w, sem_b):
    wid = lax.axis_index("s") * _NC + lax.axis_index("c")
    base = wid * _BPW
    pltpu.sync_copy(uidx_hbm.at[wid], idx_u)
    pltpu.sync_copy(midx_hbm.at[wid], idx_m)
    hb = []
    for j in range(_CH):
        hb.append(pltpu.async_copy(ubias_hbm.at[idx_u.at[j]],
                                   brows_u.at[pl.ds(j * 128, 128)], sem_b))
        hb.append(pltpu.async_copy(mbias_hbm.at[idx_m.at[j]],
                                   brows_m.at[pl.ds(j * 128, 128)], sem_b))
    hg = [pltpu.async_copy(uemb_hbm.at[idx_u.at[j]],
                           rows.at[pl.ds(j * 128, 128)], sem_g.at[j])
          for j in range(_CH)]
    hw = []
    for j in range(_CH):
        hg[j].wait()
        hw.append(pltpu.async_copy(
            rows.at[pl.ds(j * 128, 128)],
            ue_out.at[pl.ds(base + j * 128, 128)], sem_w.at[j]))
    for j in range(_CH):
        hw[j].wait()
        hg.append(pltpu.async_copy(memb_hbm.at[idx_m.at[j]],
                                   rows.at[pl.ds(j * 128, 128)], sem_g.at[j]))
    for j in range(_CH):
        hg[_CH + j].wait()
        hw.append(pltpu.async_copy(
            rows.at[pl.ds(j * 128, 128)],
            me_out.at[pl.ds(base + j * 128, 128)], sem_w.at[j]))
    for h in hb:
        h.wait()
    pltpu.sync_copy(brows_u, ub_out.at[pl.ds(base, _BPW)])
    pltpu.sync_copy(brows_m, mb_out.at[pl.ds(base, _BPW)])
    for j in range(_CH):
        hw[_CH + j].wait()


def _sc_gather(uidx, midx, uemb, memb, ubias, mbias):
    mesh = plsc.VectorSubcoreMesh(core_axis_name="c", subcore_axis_name="s")
    fn = pl.kernel(
        _sc_gather_body,
        mesh=mesh,
        out_type=(
            jax.ShapeDtypeStruct((_B, _D), jnp.float32),
            jax.ShapeDtypeStruct((_B, _D), jnp.float32),
            jax.ShapeDtypeStruct((_B,), jnp.float32),
            jax.ShapeDtypeStruct((_B,), jnp.float32),
        ),
        scratch_types=[
            pltpu.VMEM((_CH, 128), jnp.int32),
            pltpu.VMEM((_CH, 128), jnp.int32),
            pltpu.VMEM((_BPW, _D), jnp.float32),
            pltpu.VMEM((_BPW,), jnp.float32),
            pltpu.VMEM((_BPW,), jnp.float32),
            pltpu.SemaphoreType.DMA((_CH,)),
            pltpu.SemaphoreType.DMA((_CH,)),
            pltpu.SemaphoreType.DMA,
        ],
    )
    return fn(uidx, midx, uemb, memb, ubias, mbias)


def _affine(s_sum, s_sq, g, be):
    mu = s_sum * (1.0 / _B)
    var = s_sq * (1.0 / _B) - mu * mu
    a = g * lax.rsqrt(var + _EPS)
    return a, be - mu * a


def _colstats(y, ones_row):
    s = jnp.dot(ones_row, y, preferred_element_type=jnp.float32)
    sq = jnp.dot(ones_row, y * y, preferred_element_type=jnp.float32)
    return jnp.concatenate([s, sq], axis=0)


def _acc(ref, i, val):
    @pl.when(i == 0)
    def _():
        ref[...] = val

    @pl.when(i > 0)
    def _():
        ref[...] += val


def _fused_body(ue_ref, me_ref, gu_ref, bu_ref, gm_ref, bm_ref,
                w1t_ref, b1_ref, wrt_ref, sc_ref, ub_ref, mb_ref,
                g1_ref, be1_ref, w2t_ref, b2_ref,
                g2_ref, be2_ref, w3t_ref, b3_ref,
                g3_ref, be3_ref, wft_ref, bf_ref,
                out_ref,
                y_s, res_s,
                s0_s, s1_s, s2_s, s3_s):
    p = pl.program_id(0)
    i = pl.program_id(1)
    rows = pl.ds(i * _BLK, _BLK)
    ones_row = jnp.ones((1, _BLK), jnp.float32)

    @pl.when(p == 0)
    def _phase0():
        st = jnp.concatenate([_colstats(ue_ref[...], ones_row),
                              _colstats(me_ref[...], ones_row)], axis=0)
        _acc(s0_s, i, st)

    @pl.when(p == 1)
    def _phase1():
        s0 = s0_s[...]
        au, cu = _affine(s0[0:1], s0[1:2], gu_ref[...], bu_ref[...])
        am, cm = _affine(s0[2:3], s0[3:4], gm_ref[...], bm_ref[...])
        comb = jnp.concatenate([ue_ref[...] * au + cu,
                                me_ref[...] * am + cm], axis=1)
        y1 = jnp.dot(comb, w1t_ref[...],
                     preferred_element_type=jnp.float32) + b1_ref[...]
        y_s[rows, :] = y1
        res_s[rows, :] = (jnp.dot(comb, wrt_ref[...],
                                  preferred_element_type=jnp.float32)
                          + sc_ref[0, 0] + ub_ref[...] + mb_ref[...])
        _acc(s1_s, i, _colstats(y1, ones_row))

    @pl.when(p == 2)
    def _phase2():
        s1 = s1_s[...]
        a, c = _affine(s1[0:1], s1[1:2], g1_ref[...], be1_ref[...])
        x = jnp.maximum(y_s[rows, :] * a + c, 0.0)
        y2 = jnp.dot(x, w2t_ref[...],
                     preferred_element_type=jnp.float32) + b2_ref[...]
        y_s[rows, 0:256] = y2
        _acc(s2_s, i, _colstats(y2, ones_row))

    @pl.when(p == 3)
    def _phase3():
        s2 = s2_s[...]
        a, c = _affine(s2[0:1], s2[1:2], g2_ref[...], be2_ref[...])
        x = jnp.maximum(y_s[rows, 0:256] * a + c, 0.0)
        y3 = jnp.dot(x, w3t_ref[...],
                     preferred_element_type=jnp.float32) + b3_ref[...]
        y_s[rows, 0:128] = y3
        _acc(s3_s, i, _colstats(y3, ones_row))

    @pl.when(p == 4)
    def _phase4():
        s3 = s3_s[...]
        a, c = _affine(s3[0:1], s3[1:2], g3_ref[...], be3_ref[...])
        x = jnp.maximum(y_s[rows, 0:128] * a + c, 0.0)
        out_ref[...] = (jnp.dot(x, wft_ref[...],
                                preferred_element_type=jnp.float32)
                        + bf_ref[0, 0] + res_s[rows, :])


def _emb_spec():
    return pl.BlockSpec((_BLK, _D), lambda p, i: (jnp.where(p <= 1, i, 0), 0))


def _bias_spec():
    return pl.BlockSpec((_BLK, 1), lambda p, i: (jnp.where(p == 1, i, 0), 0))


def _full_spec(shape):
    nd = len(shape)
    return pl.BlockSpec(shape, lambda p, i: (0,) * nd)


def kernel(users, movies, user_emb, movie_emb, user_bias_t, movie_bias_t,
           global_bias, gu, bu, gm, bm, W1, b1, g1, be1, W2, b2, g2, be2,
           W3, b3, g3, be3, Wf, bf, Wr, br):
    uidx = users.astype(jnp.int32).reshape(_NW, _CH, 128)
    midx = movies.astype(jnp.int32).reshape(_NW, _CH, 128)
    ue, me, ubg, mbg = _sc_gather(uidx, midx, user_emb, movie_emb,
                                  user_bias_t.reshape(-1),
                                  movie_bias_t.reshape(-1))

    f32 = jnp.float32
    scalar_c = (global_bias + br).reshape(1, 1)
    out = pl.pallas_call(
        _fused_body,
        grid=(5, _NBLK),
        in_specs=[_emb_spec(), _emb_spec(),
                  _full_spec((1, _D)), _full_spec((1, _D)),
                  _full_spec((1, _D)), _full_spec((1, _D)),
                  _full_spec((2 * _D, 512)), _full_spec((1, 512)),
                  _full_spec((2 * _D, 1)), _full_spec((1, 1)),
                  _bias_spec(), _bias_spec(),
                  _full_spec((1, 512)), _full_spec((1, 512)),
                  _full_spec((512, 256)), _full_spec((1, 256)),
                  _full_spec((1, 256)), _full_spec((1, 256)),
                  _full_spec((256, _D)), _full_spec((1, _D)),
                  _full_spec((1, _D)), _full_spec((1, _D)),
                  _full_spec((_D, 1)), _full_spec((1, 1))],
        out_specs=pl.BlockSpec((_BLK, 1),
                               lambda p, i: (jnp.where(p == 4, i, 0), 0)),
        out_shape=jax.ShapeDtypeStruct((_B, 1), f32),
        scratch_shapes=[
            pltpu.VMEM((_B, 512), f32),
            pltpu.VMEM((_B, 1), f32),
            pltpu.VMEM((4, _D), f32),
            pltpu.VMEM((2, 512), f32),
            pltpu.VMEM((2, 256), f32),
            pltpu.VMEM((2, _D), f32),
        ],
        compiler_params=pltpu.CompilerParams(
            vmem_limit_bytes=110 * 1024 * 1024,
        ),
    )(ue, me, gu.reshape(1, -1), bu.reshape(1, -1),
      gm.reshape(1, -1), bm.reshape(1, -1), W1.T, b1.reshape(1, -1),
      Wr.T, scalar_c, ubg.reshape(-1, 1), mbg.reshape(-1, 1),
      g1.reshape(1, -1), be1.reshape(1, -1), W2.T, b2.reshape(1, -1),
      g2.reshape(1, -1), be2.reshape(1, -1), W3.T, b3.reshape(1, -1),
      g3.reshape(1, -1), be3.reshape(1, -1), Wf.T, bf.reshape(1, 1))

    return out.reshape(_B)


# speedup vs baseline: 1.2146x; 1.2108x over previous
import functools

import jax
import jax.numpy as jnp
from jax import lax
from jax.experimental import pallas as pl
from jax.experimental.pallas import tpu as pltpu
from jax.experimental.pallas import tpu_sc as plsc

_B = 16384
_D = 128
_NC = 2
_NS = 16
_NW = _NC * _NS
_BPW = _B // _NW
_CH = _BPW // 128
_EPS = 1e-5

_BLK = 2048
_NBLK = _B // _BLK


def _sc_gather_body(uidx_hbm, midx_hbm, uemb_hbm, memb_hbm, ubias_hbm,
                    mbias_hbm, ue_out, me_out, ub_out, mb_out,
                    idx_u, idx_m, rows, brows_u, brows_m,
                    sem_g, sem_w, sem_b):
    wid = lax.axis_index("s") * _NC + lax.axis_index("c")
    base = wid * _BPW
    pltpu.sync_copy(uidx_hbm.at[wid], idx_u)
    pltpu.sync_copy(midx_hbm.at[wid], idx_m)
    hb = []
    for j in range(_CH):
        hb.append(pltpu.async_copy(ubias_hbm.at[idx_u.at[j]],
                                   brows_u.at[pl.ds(j * 128, 128)], sem_b))
        hb.append(pltpu.async_copy(mbias_hbm.at[idx_m.at[j]],
                                   brows_m.at[pl.ds(j * 128, 128)], sem_b))
    hg = [pltpu.async_copy(uemb_hbm.at[idx_u.at[j]],
                           rows.at[pl.ds(j * 128, 128)], sem_g.at[j])
          for j in range(_CH)]
    hw = []
    for j in range(_CH):
        hg[j].wait()
        hw.append(pltpu.async_copy(
            rows.at[pl.ds(j * 128, 128)],
            ue_out.at[pl.ds(base + j * 128, 128)], sem_w.at[j]))
    for j in range(_CH):
        hw[j].wait()
        hg.append(pltpu.async_copy(memb_hbm.at[idx_m.at[j]],
                                   rows.at[pl.ds(j * 128, 128)], sem_g.at[j]))
    for j in range(_CH):
        hg[_CH + j].wait()
        hw.append(pltpu.async_copy(
            rows.at[pl.ds(j * 128, 128)],
            me_out.at[pl.ds(base + j * 128, 128)], sem_w.at[j]))
    for h in hb:
        h.wait()
    pltpu.sync_copy(brows_u, ub_out.at[pl.ds(base, _BPW)])
    pltpu.sync_copy(brows_m, mb_out.at[pl.ds(base, _BPW)])
    for j in range(_CH):
        hw[_CH + j].wait()


def _sc_gather(uidx, midx, uemb, memb, ubias, mbias):
    mesh = plsc.VectorSubcoreMesh(core_axis_name="c", subcore_axis_name="s")
    fn = pl.kernel(
        _sc_gather_body,
        mesh=mesh,
        out_type=(
            jax.ShapeDtypeStruct((_B, _D), jnp.float32),
            jax.ShapeDtypeStruct((_B, _D), jnp.float32),
            jax.ShapeDtypeStruct((_B,), jnp.float32),
            jax.ShapeDtypeStruct((_B,), jnp.float32),
        ),
        scratch_types=[
            pltpu.VMEM((_CH, 128), jnp.int32),
            pltpu.VMEM((_CH, 128), jnp.int32),
            pltpu.VMEM((_BPW, _D), jnp.float32),
            pltpu.VMEM((_BPW,), jnp.float32),
            pltpu.VMEM((_BPW,), jnp.float32),
            pltpu.SemaphoreType.DMA((_CH,)),
            pltpu.SemaphoreType.DMA((_CH,)),
            pltpu.SemaphoreType.DMA,
        ],
    )
    return fn(uidx, midx, uemb, memb, ubias, mbias)


def _affine(s_sum, s_sq, g, be):
    mu = s_sum * (1.0 / _B)
    var = s_sq * (1.0 / _B) - mu * mu
    a = g * lax.rsqrt(var + _EPS)
    return a, be - mu * a


def _colstats(y, ones_row):
    s = jnp.dot(ones_row, y, preferred_element_type=jnp.float32)
    sq = jnp.dot(ones_row, y * y, preferred_element_type=jnp.float32)
    return jnp.concatenate([s, sq], axis=0)


def _acc(ref, i, val):
    @pl.when(i == 0)
    def _():
        ref[...] = val

    @pl.when(i > 0)
    def _():
        ref[...] += val


def _fused_body(ue_ref, me_ref, gu_ref, bu_ref, gm_ref, bm_ref,
                w1t_ref, b1_ref, wr_ref, sc_ref, ub_ref, mb_ref,
                g1_ref, be1_ref, w2t_ref, b2_ref,
                g2_ref, be2_ref, w3t_ref, b3_ref,
                g3_ref, be3_ref, wf_ref, bf_ref,
                out_ref,
                y_s, res_s,
                s0_s, s1_s, s2_s, s3_s):
    p = pl.program_id(0)
    i = pl.program_id(1)
    rows = pl.ds(i * _BLK, _BLK)
    ones_row = jnp.ones((1, _BLK), jnp.float32)

    @pl.when(p == 0)
    def _phase0():
        st = jnp.concatenate([_colstats(ue_ref[...], ones_row),
                              _colstats(me_ref[...], ones_row)], axis=0)
        _acc(s0_s, i, st)

    @pl.when(p == 1)
    def _phase1():
        s0 = s0_s[...]
        au, cu = _affine(s0[0:1], s0[1:2], gu_ref[...], bu_ref[...])
        am, cm = _affine(s0[2:3], s0[3:4], gm_ref[...], bm_ref[...])
        comb = jnp.concatenate([ue_ref[...] * au + cu,
                                me_ref[...] * am + cm], axis=1)
        y1 = jnp.dot(comb, w1t_ref[...],
                     preferred_element_type=jnp.float32) + b1_ref[...]
        y_s[rows, :] = y1
        res_s[rows] = (jnp.sum(comb * wr_ref[...], axis=1)
                       + sc_ref[0, 0] + ub_ref[...] + mb_ref[...])

        _acc(s1_s, i, _colstats(y1, ones_row))

    @pl.when(p == 2)
    def _phase2():
        s1 = s1_s[...]
        a, c = _affine(s1[0:1], s1[1:2], g1_ref[...], be1_ref[...])
        x = jnp.maximum(y_s[rows, :] * a + c, 0.0)
        y2 = jnp.dot(x, w2t_ref[...],
                     preferred_element_type=jnp.float32) + b2_ref[...]
        y_s[rows, 0:256] = y2
        _acc(s2_s, i, _colstats(y2, ones_row))

    @pl.when(p == 3)
    def _phase3():
        s2 = s2_s[...]
        a, c = _affine(s2[0:1], s2[1:2], g2_ref[...], be2_ref[...])
        x = jnp.maximum(y_s[rows, 0:256] * a + c, 0.0)
        y3 = jnp.dot(x, w3t_ref[...],
                     preferred_element_type=jnp.float32) + b3_ref[...]
        y_s[rows, 0:128] = y3
        _acc(s3_s, i, _colstats(y3, ones_row))

    @pl.when(p == 4)
    def _phase4():
        s3 = s3_s[...]
        a, c = _affine(s3[0:1], s3[1:2], g3_ref[...], be3_ref[...])
        x = jnp.maximum(y_s[rows, 0:128] * a + c, 0.0)
        out_ref[...] = (jnp.sum(x * wf_ref[...], axis=1)
                        + bf_ref[0, 0] + res_s[rows])


def _emb_spec():
    return pl.BlockSpec((_BLK, _D), lambda p, i: (jnp.where(p <= 1, i, 0), 0))


def _bias_spec():
    return pl.BlockSpec((_BLK,), lambda p, i: (jnp.where(p == 1, i, 0),))


def _full_spec(shape):
    nd = len(shape)
    return pl.BlockSpec(shape, lambda p, i: (0,) * nd)


def kernel(users, movies, user_emb, movie_emb, user_bias_t, movie_bias_t,
           global_bias, gu, bu, gm, bm, W1, b1, g1, be1, W2, b2, g2, be2,
           W3, b3, g3, be3, Wf, bf, Wr, br):
    uidx = users.astype(jnp.int32).reshape(_NW, _CH, 128)
    midx = movies.astype(jnp.int32).reshape(_NW, _CH, 128)
    ue, me, ubg, mbg = _sc_gather(uidx, midx, user_emb, movie_emb,
                                  user_bias_t.reshape(-1),
                                  movie_bias_t.reshape(-1))

    f32 = jnp.float32
    scalar_c = (global_bias + br).reshape(1, 1)
    out = pl.pallas_call(
        _fused_body,
        grid=(5, _NBLK),
        in_specs=[_emb_spec(), _emb_spec(),
                  _full_spec((1, _D)), _full_spec((1, _D)),
                  _full_spec((1, _D)), _full_spec((1, _D)),
                  _full_spec((2 * _D, 512)), _full_spec((1, 512)),
                  _full_spec((1, 2 * _D)), _full_spec((1, 1)),
                  _bias_spec(), _bias_spec(),
                  _full_spec((1, 512)), _full_spec((1, 512)),
                  _full_spec((512, 256)), _full_spec((1, 256)),
                  _full_spec((1, 256)), _full_spec((1, 256)),
                  _full_spec((256, _D)), _full_spec((1, _D)),
                  _full_spec((1, _D)), _full_spec((1, _D)),
                  _full_spec((1, _D)), _full_spec((1, 1))],
        out_specs=pl.BlockSpec((_BLK,),
                               lambda p, i: (jnp.where(p == 4, i, 0),)),
        out_shape=jax.ShapeDtypeStruct((_B,), f32),
        scratch_shapes=[
            pltpu.VMEM((_B, 512), f32),
            pltpu.VMEM((_B,), f32),
            pltpu.VMEM((4, _D), f32),
            pltpu.VMEM((2, 512), f32),
            pltpu.VMEM((2, 256), f32),
            pltpu.VMEM((2, _D), f32),
        ],
        compiler_params=pltpu.CompilerParams(
            vmem_limit_bytes=110 * 1024 * 1024,
        ),
    )(ue, me, gu.reshape(1, -1), bu.reshape(1, -1),
      gm.reshape(1, -1), bm.reshape(1, -1), W1.T, b1.reshape(1, -1),
      Wr, scalar_c, ubg, mbg,
      g1.reshape(1, -1), be1.reshape(1, -1), W2.T, b2.reshape(1, -1),
      g2.reshape(1, -1), be2.reshape(1, -1), W3.T, b3.reshape(1, -1),
      g3.reshape(1, -1), be3.reshape(1, -1), Wf, bf.reshape(1, 1))

    return out


# speedup vs baseline: 1.3284x; 1.0937x over previous
import functools

import jax
import jax.numpy as jnp
from jax import lax
from jax.experimental import pallas as pl
from jax.experimental.pallas import tpu as pltpu
from jax.experimental.pallas import tpu_sc as plsc

_B = 16384
_D = 128
_NC = 2
_NS = 16
_NW = _NC * _NS
_BPW = _B // _NW
_CH = _BPW // 128
_EPS = 1e-5

_BLK = 4096
_NBLK = _B // _BLK


def _sc_gather_body(uidx_hbm, midx_hbm, uemb_hbm, memb_hbm, ubias_hbm,
                    mbias_hbm, ue_out, me_out, ub_out, mb_out,
                    idx_u, idx_m, rows, brows_u, brows_m,
                    sem_g, sem_w, sem_b):
    wid = lax.axis_index("s") * _NC + lax.axis_index("c")
    base = wid * _BPW
    pltpu.sync_copy(uidx_hbm.at[wid], idx_u)
    pltpu.sync_copy(midx_hbm.at[wid], idx_m)
    hb = []
    for j in range(_CH):
        hb.append(pltpu.async_copy(ubias_hbm.at[idx_u.at[j]],
                                   brows_u.at[pl.ds(j * 128, 128)], sem_b))
        hb.append(pltpu.async_copy(mbias_hbm.at[idx_m.at[j]],
                                   brows_m.at[pl.ds(j * 128, 128)], sem_b))
    hg = [pltpu.async_copy(uemb_hbm.at[idx_u.at[j]],
                           rows.at[pl.ds(j * 128, 128)], sem_g.at[j])
          for j in range(_CH)]
    hw = []
    for j in range(_CH):
        hg[j].wait()
        hw.append(pltpu.async_copy(
            rows.at[pl.ds(j * 128, 128)],
            ue_out.at[pl.ds(base + j * 128, 128)], sem_w.at[j]))
    for j in range(_CH):
        hw[j].wait()
        hg.append(pltpu.async_copy(memb_hbm.at[idx_m.at[j]],
                                   rows.at[pl.ds(j * 128, 128)], sem_g.at[j]))
    for j in range(_CH):
        hg[_CH + j].wait()
        hw.append(pltpu.async_copy(
            rows.at[pl.ds(j * 128, 128)],
            me_out.at[pl.ds(base + j * 128, 128)], sem_w.at[j]))
    for h in hb:
        h.wait()
    pltpu.sync_copy(brows_u, ub_out.at[pl.ds(base, _BPW)])
    pltpu.sync_copy(brows_m, mb_out.at[pl.ds(base, _BPW)])
    for j in range(_CH):
        hw[_CH + j].wait()


def _sc_gather(uidx, midx, uemb, memb, ubias, mbias):
    mesh = plsc.VectorSubcoreMesh(core_axis_name="c", subcore_axis_name="s")
    fn = pl.kernel(
        _sc_gather_body,
        mesh=mesh,
        out_type=(
            jax.ShapeDtypeStruct((_B, _D), jnp.float32),
            jax.ShapeDtypeStruct((_B, _D), jnp.float32),
            jax.ShapeDtypeStruct((_B,), jnp.float32),
            jax.ShapeDtypeStruct((_B,), jnp.float32),
        ),
        scratch_types=[
            pltpu.VMEM((_CH, 128), jnp.int32),
            pltpu.VMEM((_CH, 128), jnp.int32),
            pltpu.VMEM((_BPW, _D), jnp.float32),
            pltpu.VMEM((_BPW,), jnp.float32),
            pltpu.VMEM((_BPW,), jnp.float32),
            pltpu.SemaphoreType.DMA((_CH,)),
            pltpu.SemaphoreType.DMA((_CH,)),
            pltpu.SemaphoreType.DMA,
        ],
    )
    return fn(uidx, midx, uemb, memb, ubias, mbias)


def _affine(s_sum, s_sq, g, be):
    mu = s_sum * (1.0 / _B)
    var = s_sq * (1.0 / _B) - mu * mu
    a = g * lax.rsqrt(var + _EPS)
    return a, be - mu * a


def _colstats(y, ones_row):
    s = jnp.dot(ones_row, y, preferred_element_type=jnp.float32)
    sq = jnp.dot(ones_row, y * y, preferred_element_type=jnp.float32)
    return jnp.concatenate([s, sq], axis=0)


def _acc(ref, i, val):
    @pl.when(i == 0)
    def _():
        ref[...] = val

    @pl.when(i > 0)
    def _():
        ref[...] += val


def _fused_body(ue_ref, me_ref, gu_ref, bu_ref, gm_ref, bm_ref,
                w1t_ref, b1_ref, wr_ref, sc_ref, ub_ref, mb_ref,
                g1_ref, be1_ref, w2t_ref, b2_ref,
                g2_ref, be2_ref, w3t_ref, b3_ref,
                g3_ref, be3_ref, wf_ref, bf_ref,
                out_ref,
                y_s, res_s,
                s0_s, s1_s, s2_s, s3_s):
    p = pl.program_id(0)
    i = pl.program_id(1)
    rows = pl.ds(i * _BLK, _BLK)
    ones_row = jnp.ones((1, _BLK), jnp.float32)

    @pl.when(p == 0)
    def _phase0():
        st = jnp.concatenate([_colstats(ue_ref[...], ones_row),
                              _colstats(me_ref[...], ones_row)], axis=0)
        _acc(s0_s, i, st)

    @pl.when(p == 1)
    def _phase1():
        s0 = s0_s[...]
        au, cu = _affine(s0[0:1], s0[1:2], gu_ref[...], bu_ref[...])
        am, cm = _affine(s0[2:3], s0[3:4], gm_ref[...], bm_ref[...])
        comb = jnp.concatenate([ue_ref[...] * au + cu,
                                me_ref[...] * am + cm], axis=1)
        y1 = jnp.dot(comb, w1t_ref[...],
                     preferred_element_type=jnp.float32) + b1_ref[...]
        y_s[rows, :] = y1
        res_s[rows] = (jnp.sum(comb * wr_ref[...], axis=1)
                       + sc_ref[0, 0] + ub_ref[...] + mb_ref[...])

        _acc(s1_s, i, _colstats(y1, ones_row))

    @pl.when(p == 2)
    def _phase2():
        s1 = s1_s[...]
        a, c = _affine(s1[0:1], s1[1:2], g1_ref[...], be1_ref[...])
        x = jnp.maximum(y_s[rows, :] * a + c, 0.0)
        y2 = jnp.dot(x, w2t_ref[...],
                     preferred_element_type=jnp.float32) + b2_ref[...]
        y_s[rows, 0:256] = y2
        _acc(s2_s, i, _colstats(y2, ones_row))

    @pl.when(p == 3)
    def _phase3():
        s2 = s2_s[...]
        a, c = _affine(s2[0:1], s2[1:2], g2_ref[...], be2_ref[...])
        x = jnp.maximum(y_s[rows, 0:256] * a + c, 0.0)
        y3 = jnp.dot(x, w3t_ref[...],
                     preferred_element_type=jnp.float32) + b3_ref[...]
        y_s[rows, 0:128] = y3
        _acc(s3_s, i, _colstats(y3, ones_row))

    @pl.when(p == 4)
    def _phase4():
        s3 = s3_s[...]
        a, c = _affine(s3[0:1], s3[1:2], g3_ref[...], be3_ref[...])
        x = jnp.maximum(y_s[rows, 0:128] * a + c, 0.0)
        out_ref[...] = (jnp.sum(x * wf_ref[...], axis=1)
                        + bf_ref[0, 0] + res_s[rows])


def _emb_spec():
    return pl.BlockSpec((_BLK, _D), lambda p, i: (jnp.where(p <= 1, i, 0), 0))


def _bias_spec():
    return pl.BlockSpec((_BLK,), lambda p, i: (jnp.where(p == 1, i, 0),))


def _full_spec(shape):
    nd = len(shape)
    return pl.BlockSpec(shape, lambda p, i: (0,) * nd)


def kernel(users, movies, user_emb, movie_emb, user_bias_t, movie_bias_t,
           global_bias, gu, bu, gm, bm, W1, b1, g1, be1, W2, b2, g2, be2,
           W3, b3, g3, be3, Wf, bf, Wr, br):
    uidx = users.astype(jnp.int32).reshape(_NW, _CH, 128)
    midx = movies.astype(jnp.int32).reshape(_NW, _CH, 128)
    ue, me, ubg, mbg = _sc_gather(uidx, midx, user_emb, movie_emb,
                                  user_bias_t.reshape(-1),
                                  movie_bias_t.reshape(-1))

    f32 = jnp.float32
    scalar_c = (global_bias + br).reshape(1, 1)
    out = pl.pallas_call(
        _fused_body,
        grid=(5, _NBLK),
        in_specs=[_emb_spec(), _emb_spec(),
                  _full_spec((1, _D)), _full_spec((1, _D)),
                  _full_spec((1, _D)), _full_spec((1, _D)),
                  _full_spec((2 * _D, 512)), _full_spec((1, 512)),
                  _full_spec((1, 2 * _D)), _full_spec((1, 1)),
                  _bias_spec(), _bias_spec(),
                  _full_spec((1, 512)), _full_spec((1, 512)),
                  _full_spec((512, 256)), _full_spec((1, 256)),
                  _full_spec((1, 256)), _full_spec((1, 256)),
                  _full_spec((256, _D)), _full_spec((1, _D)),
                  _full_spec((1, _D)), _full_spec((1, _D)),
                  _full_spec((1, _D)), _full_spec((1, 1))],
        out_specs=pl.BlockSpec((_BLK,),
                               lambda p, i: (jnp.where(p == 4, i, 0),)),
        out_shape=jax.ShapeDtypeStruct((_B,), f32),
        scratch_shapes=[
            pltpu.VMEM((_B, 512), f32),
            pltpu.VMEM((_B,), f32),
            pltpu.VMEM((4, _D), f32),
            pltpu.VMEM((2, 512), f32),
            pltpu.VMEM((2, 256), f32),
            pltpu.VMEM((2, _D), f32),
        ],
        compiler_params=pltpu.CompilerParams(
            vmem_limit_bytes=110 * 1024 * 1024,
        ),
    )(ue, me, gu.reshape(1, -1), bu.reshape(1, -1),
      gm.reshape(1, -1), bm.reshape(1, -1), W1.T, b1.reshape(1, -1),
      Wr, scalar_c, ubg, mbg,
      g1.reshape(1, -1), be1.reshape(1, -1), W2.T, b2.reshape(1, -1),
      g2.reshape(1, -1), be2.reshape(1, -1), W3.T, b3.reshape(1, -1),
      g3.reshape(1, -1), be3.reshape(1, -1), Wf, bf.reshape(1, 1))

    return out


# speedup vs baseline: 1.4089x; 1.0606x over previous
import functools

import jax
import jax.numpy as jnp
from jax import lax
from jax.experimental import pallas as pl
from jax.experimental.pallas import tpu as pltpu
from jax.experimental.pallas import tpu_sc as plsc

_B = 16384
_D = 128
_NC = 2
_NS = 16
_NW = _NC * _NS
_BPW = _B // _NW
_CH = _BPW // 128
_EPS = 1e-5

_BLK = 4096
_NBLK = _B // _BLK


def _chunk_stats(rows, j, sums, sqs):
    def body(r, carry):
        out = []
        for c in range(_D // 16):
            v = rows[j * 128 + r, pl.ds(c * 16, 16)]
            out.append(carry[c] + v)
            out.append(carry[_D // 16 + c] + v * v)
        return tuple(out[::2]) + tuple(out[1::2])

    carry = lax.fori_loop(0, 128, body, tuple(sums) + tuple(sqs))
    return list(carry[:_D // 16]), list(carry[_D // 16:])


def _sc_gather_body(uidx_hbm, midx_hbm, uemb_hbm, memb_hbm, ubias_hbm,
                    mbias_hbm, ue_out, me_out, ub_out, mb_out, st_out,
                    idx_u, idx_m, rows, brows_u, brows_m, st_v,
                    sem_g, sem_w, sem_b):
    wid = lax.axis_index("s") * _NC + lax.axis_index("c")
    base = wid * _BPW
    pltpu.sync_copy(uidx_hbm.at[wid], idx_u)
    pltpu.sync_copy(midx_hbm.at[wid], idx_m)
    hb = []
    for j in range(_CH):
        hb.append(pltpu.async_copy(ubias_hbm.at[idx_u.at[j]],
                                   brows_u.at[pl.ds(j * 128, 128)], sem_b))
        hb.append(pltpu.async_copy(mbias_hbm.at[idx_m.at[j]],
                                   brows_m.at[pl.ds(j * 128, 128)], sem_b))
    zeros = [jnp.zeros((16,), jnp.float32)] * (_D // 16)
    hg = [pltpu.async_copy(uemb_hbm.at[idx_u.at[j]],
                           rows.at[pl.ds(j * 128, 128)], sem_g.at[j])
          for j in range(_CH)]
    hw = []
    su, qu = zeros, zeros
    for j in range(_CH):
        hg[j].wait()
        hw.append(pltpu.async_copy(
            rows.at[pl.ds(j * 128, 128)],
            ue_out.at[pl.ds(base + j * 128, 128)], sem_w.at[j]))
        su, qu = _chunk_stats(rows, j, su, qu)
        hw[j].wait()
        hg.append(pltpu.async_copy(memb_hbm.at[idx_m.at[j]],
                                   rows.at[pl.ds(j * 128, 128)], sem_g.at[j]))
    sm, qm = zeros, zeros
    for j in range(_CH):
        hg[_CH + j].wait()
        hw.append(pltpu.async_copy(
            rows.at[pl.ds(j * 128, 128)],
            me_out.at[pl.ds(base + j * 128, 128)], sem_w.at[j]))
        sm, qm = _chunk_stats(rows, j, sm, qm)
    for c in range(_D // 16):
        st_v[0, pl.ds(c * 16, 16)] = su[c]
        st_v[1, pl.ds(c * 16, 16)] = qu[c]
        st_v[2, pl.ds(c * 16, 16)] = sm[c]
        st_v[3, pl.ds(c * 16, 16)] = qm[c]
    pltpu.sync_copy(st_v, st_out.at[wid])
    for h in hb:
        h.wait()
    pltpu.sync_copy(brows_u, ub_out.at[pl.ds(base, _BPW)])
    pltpu.sync_copy(brows_m, mb_out.at[pl.ds(base, _BPW)])
    for j in range(_CH):
        hw[_CH + j].wait()


def _sc_gather(uidx, midx, uemb, memb, ubias, mbias):
    mesh = plsc.VectorSubcoreMesh(core_axis_name="c", subcore_axis_name="s")
    fn = pl.kernel(
        _sc_gather_body,
        mesh=mesh,
        out_type=(
            jax.ShapeDtypeStruct((_B, _D), jnp.float32),
            jax.ShapeDtypeStruct((_B, _D), jnp.float32),
            jax.ShapeDtypeStruct((_B,), jnp.float32),
            jax.ShapeDtypeStruct((_B,), jnp.float32),
            jax.ShapeDtypeStruct((_NW, 4, _D), jnp.float32),
        ),
        scratch_types=[
            pltpu.VMEM((_CH, 128), jnp.int32),
            pltpu.VMEM((_CH, 128), jnp.int32),
            pltpu.VMEM((_BPW, _D), jnp.float32),
            pltpu.VMEM((_BPW,), jnp.float32),
            pltpu.VMEM((_BPW,), jnp.float32),
            pltpu.VMEM((4, _D), jnp.float32),
            pltpu.SemaphoreType.DMA((_CH,)),
            pltpu.SemaphoreType.DMA((_CH,)),
            pltpu.SemaphoreType.DMA,
        ],
    )
    return fn(uidx, midx, uemb, memb, ubias, mbias)


def _affine(s_sum, s_sq, g, be):
    mu = s_sum * (1.0 / _B)
    var = s_sq * (1.0 / _B) - mu * mu
    a = g * lax.rsqrt(var + _EPS)
    return a, be - mu * a


def _colstats(y, ones_row):
    s = jnp.dot(ones_row, y, preferred_element_type=jnp.float32)
    sq = jnp.dot(ones_row, y * y, preferred_element_type=jnp.float32)
    return jnp.concatenate([s, sq], axis=0)


def _acc(ref, i, val):
    @pl.when(i == 0)
    def _():
        ref[...] = val

    @pl.when(i > 0)
    def _():
        ref[...] += val


def _fused_body(ue_ref, me_ref, st0_ref, gu_ref, bu_ref, gm_ref, bm_ref,
                w1t_ref, b1_ref, wr_ref, sc_ref, ub_ref, mb_ref,
                g1_ref, be1_ref, w2t_ref, b2_ref,
                g2_ref, be2_ref, w3t_ref, b3_ref,
                g3_ref, be3_ref, wf_ref, bf_ref,
                out_ref,
                y_s, res_s,
                s1_s, s2_s, s3_s):
    p = pl.program_id(0)
    i = pl.program_id(1)
    rows = pl.ds(i * _BLK, _BLK)
    ones_row = jnp.ones((1, _BLK), jnp.float32)

    @pl.when(p == 0)
    def _phase1():
        s0 = jnp.sum(st0_ref[...], axis=0)
        au, cu = _affine(s0[0:1], s0[1:2], gu_ref[...], bu_ref[...])
        am, cm = _affine(s0[2:3], s0[3:4], gm_ref[...], bm_ref[...])
        comb = jnp.concatenate([ue_ref[...] * au + cu,
                                me_ref[...] * am + cm], axis=1)
        y1 = jnp.dot(comb, w1t_ref[...],
                     preferred_element_type=jnp.float32) + b1_ref[...]
        y_s[rows, :] = y1
        res_s[rows] = (jnp.sum(comb * wr_ref[...], axis=1)
                       + sc_ref[0, 0] + ub_ref[...] + mb_ref[...])

        _acc(s1_s, i, _colstats(y1, ones_row))

    @pl.when(p == 1)
    def _phase2():
        s1 = s1_s[...]
        a, c = _affine(s1[0:1], s1[1:2], g1_ref[...], be1_ref[...])
        x = jnp.maximum(y_s[rows, :] * a + c, 0.0)
        y2 = jnp.dot(x, w2t_ref[...],
                     preferred_element_type=jnp.float32) + b2_ref[...]
        y_s[rows, 0:256] = y2
        _acc(s2_s, i, _colstats(y2, ones_row))

    @pl.when(p == 2)
    def _phase3():
        s2 = s2_s[...]
        a, c = _affine(s2[0:1], s2[1:2], g2_ref[...], be2_ref[...])
        x = jnp.maximum(y_s[rows, 0:256] * a + c, 0.0)
        y3 = jnp.dot(x, w3t_ref[...],
                     preferred_element_type=jnp.float32) + b3_ref[...]
        y_s[rows, 0:128] = y3
        _acc(s3_s, i, _colstats(y3, ones_row))

    @pl.when(p == 3)
    def _phase4():
        s3 = s3_s[...]
        a, c = _affine(s3[0:1], s3[1:2], g3_ref[...], be3_ref[...])
        x = jnp.maximum(y_s[rows, 0:128] * a + c, 0.0)
        out_ref[...] = (jnp.sum(x * wf_ref[...], axis=1)
                        + bf_ref[0, 0] + res_s[rows])


def _emb_spec():
    return pl.BlockSpec((_BLK, _D), lambda p, i: (jnp.where(p == 0, i, 0), 0))


def _bias_spec():
    return pl.BlockSpec((_BLK,), lambda p, i: (jnp.where(p == 0, i, 0),))


def _full_spec(shape):
    nd = len(shape)
    return pl.BlockSpec(shape, lambda p, i: (0,) * nd)


def kernel(users, movies, user_emb, movie_emb, user_bias_t, movie_bias_t,
           global_bias, gu, bu, gm, bm, W1, b1, g1, be1, W2, b2, g2, be2,
           W3, b3, g3, be3, Wf, bf, Wr, br):
    uidx = users.astype(jnp.int32).reshape(_NW, _CH, 128)
    midx = movies.astype(jnp.int32).reshape(_NW, _CH, 128)
    ue, me, ubg, mbg, st0 = _sc_gather(uidx, midx, user_emb, movie_emb,
                                       user_bias_t.reshape(-1),
                                       movie_bias_t.reshape(-1))

    f32 = jnp.float32
    scalar_c = (global_bias + br).reshape(1, 1)
    out = pl.pallas_call(
        _fused_body,
        grid=(4, _NBLK),
        in_specs=[_emb_spec(), _emb_spec(),
                  _full_spec((_NW, 4, _D)),
                  _full_spec((1, _D)), _full_spec((1, _D)),
                  _full_spec((1, _D)), _full_spec((1, _D)),
                  _full_spec((2 * _D, 512)), _full_spec((1, 512)),
                  _full_spec((1, 2 * _D)), _full_spec((1, 1)),
                  _bias_spec(), _bias_spec(),
                  _full_spec((1, 512)), _full_spec((1, 512)),
                  _full_spec((512, 256)), _full_spec((1, 256)),
                  _full_spec((1, 256)), _full_spec((1, 256)),
                  _full_spec((256, _D)), _full_spec((1, _D)),
                  _full_spec((1, _D)), _full_spec((1, _D)),
                  _full_spec((1, _D)), _full_spec((1, 1))],
        out_specs=pl.BlockSpec((_BLK,),
                               lambda p, i: (jnp.where(p == 3, i, 0),)),
        out_shape=jax.ShapeDtypeStruct((_B,), f32),
        scratch_shapes=[
            pltpu.VMEM((_B, 512), f32),
            pltpu.VMEM((_B,), f32),
            pltpu.VMEM((2, 512), f32),
            pltpu.VMEM((2, 256), f32),
            pltpu.VMEM((2, _D), f32),
        ],
        compiler_params=pltpu.CompilerParams(
            vmem_limit_bytes=110 * 1024 * 1024,
        ),
    )(ue, me, st0, gu.reshape(1, -1), bu.reshape(1, -1),
      gm.reshape(1, -1), bm.reshape(1, -1), W1.T, b1.reshape(1, -1),
      Wr, scalar_c, ubg, mbg,
      g1.reshape(1, -1), be1.reshape(1, -1), W2.T, b2.reshape(1, -1),
      g2.reshape(1, -1), be2.reshape(1, -1), W3.T, b3.reshape(1, -1),
      g3.reshape(1, -1), be3.reshape(1, -1), Wf, bf.reshape(1, 1))

    return out


# speedup vs baseline: 1.4140x; 1.0036x over previous
import functools

import jax
import jax.numpy as jnp
from jax import lax
from jax.experimental import pallas as pl
from jax.experimental.pallas import tpu as pltpu
from jax.experimental.pallas import tpu_sc as plsc

_B = 16384
_D = 128
_NC = 2
_NS = 16
_NW = _NC * _NS
_BPW = _B // _NW
_CH = _BPW // 128
_EPS = 1e-5

_BLK = 4096
_NBLK = _B // _BLK


def _chunk_stats(rows, j, sums, sqs):
    def body(r, carry):
        out = []
        for c in range(_D // 16):
            v = rows[j * 128 + r, pl.ds(c * 16, 16)]
            out.append(carry[c] + v)
            out.append(carry[_D // 16 + c] + v * v)
        return tuple(out[::2]) + tuple(out[1::2])

    carry = lax.fori_loop(0, 128, body, tuple(sums) + tuple(sqs))
    return list(carry[:_D // 16]), list(carry[_D // 16:])


def _sc_gather_body(uidx_hbm, midx_hbm, uemb_hbm, memb_hbm, ubias_hbm,
                    mbias_hbm, ue_out, me_out, ub_out, mb_out, st_out,
                    idx_u, idx_m, rows, brows_u, brows_m, st_v,
                    sem_g, sem_w, sem_b):
    wid = lax.axis_index("s") * _NC + lax.axis_index("c")
    base = wid * _BPW
    pltpu.sync_copy(uidx_hbm.at[wid], idx_u)
    pltpu.sync_copy(midx_hbm.at[wid], idx_m)
    hb = []
    for j in range(_CH):
        hb.append(pltpu.async_copy(ubias_hbm.at[idx_u.at[j]],
                                   brows_u.at[pl.ds(j * 128, 128)], sem_b))
        hb.append(pltpu.async_copy(mbias_hbm.at[idx_m.at[j]],
                                   brows_m.at[pl.ds(j * 128, 128)], sem_b))
    zeros = [jnp.zeros((16,), jnp.float32)] * (_D // 16)
    hg = [pltpu.async_copy(uemb_hbm.at[idx_u.at[j]],
                           rows.at[pl.ds(j * 128, 128)], sem_g.at[j])
          for j in range(_CH)]
    hw = []
    su, qu = zeros, zeros
    for j in range(_CH):
        hg[j].wait()
        hw.append(pltpu.async_copy(
            rows.at[pl.ds(j * 128, 128)],
            ue_out.at[pl.ds(base + j * 128, 128)], sem_w.at[j]))
        su, qu = _chunk_stats(rows, j, su, qu)
        hw[j].wait()
        hg.append(pltpu.async_copy(memb_hbm.at[idx_m.at[j]],
                                   rows.at[pl.ds(j * 128, 128)], sem_g.at[j]))
    sm, qm = zeros, zeros
    for j in range(_CH):
        hg[_CH + j].wait()
        hw.append(pltpu.async_copy(
            rows.at[pl.ds(j * 128, 128)],
            me_out.at[pl.ds(base + j * 128, 128)], sem_w.at[j]))
        sm, qm = _chunk_stats(rows, j, sm, qm)
    for c in range(_D // 16):
        st_v[0, pl.ds(c * 16, 16)] = su[c]
        st_v[1, pl.ds(c * 16, 16)] = qu[c]
        st_v[2, pl.ds(c * 16, 16)] = sm[c]
        st_v[3, pl.ds(c * 16, 16)] = qm[c]
    pltpu.sync_copy(st_v, st_out.at[wid])
    for h in hb:
        h.wait()
    pltpu.sync_copy(brows_u, ub_out.at[pl.ds(base, _BPW)])
    pltpu.sync_copy(brows_m, mb_out.at[pl.ds(base, _BPW)])
    for j in range(_CH):
        hw[_CH + j].wait()


def _sc_gather(uidx, midx, uemb, memb, ubias, mbias):
    mesh = plsc.VectorSubcoreMesh(core_axis_name="c", subcore_axis_name="s")
    fn = pl.kernel(
        _sc_gather_body,
        mesh=mesh,
        out_type=(
            jax.ShapeDtypeStruct((_B, _D), jnp.float32),
            jax.ShapeDtypeStruct((_B, _D), jnp.float32),
            jax.ShapeDtypeStruct((_B,), jnp.float32),
            jax.ShapeDtypeStruct((_B,), jnp.float32),
            jax.ShapeDtypeStruct((_NW, 4, _D), jnp.float32),
        ),
        scratch_types=[
            pltpu.VMEM((_CH, 128), jnp.int32),
            pltpu.VMEM((_CH, 128), jnp.int32),
            pltpu.VMEM((_BPW, _D), jnp.float32),
            pltpu.VMEM((_BPW,), jnp.float32),
            pltpu.VMEM((_BPW,), jnp.float32),
            pltpu.VMEM((4, _D), jnp.float32),
            pltpu.SemaphoreType.DMA((_CH,)),
            pltpu.SemaphoreType.DMA((_CH,)),
            pltpu.SemaphoreType.DMA,
        ],
    )
    return fn(uidx, midx, uemb, memb, ubias, mbias)


def _affine(s_sum, s_sq, g, be):
    mu = s_sum * (1.0 / _B)
    var = s_sq * (1.0 / _B) - mu * mu
    a = g * lax.rsqrt(var + _EPS)
    return a, be - mu * a


def _colstats(y, ones_row):
    s = jnp.dot(ones_row, y, preferred_element_type=jnp.float32)
    sq = jnp.dot(ones_row, y * y, preferred_element_type=jnp.float32)
    return jnp.concatenate([s, sq], axis=0)


def _acc(ref, i, val):
    @pl.when(i == 0)
    def _():
        ref[...] = val

    @pl.when(i > 0)
    def _():
        ref[...] += val


def _fused_body(ue_ref, me_ref, st0_ref, gu_ref, bu_ref, gm_ref, bm_ref,
                w1t_ref, b1_ref, wr_ref, sc_ref, ub_ref, mb_ref,
                g1_ref, be1_ref, w2t_ref, b2_ref,
                g2_ref, be2_ref, w3t_ref, b3_ref,
                g3_ref, be3_ref, wf_ref, bf_ref,
                out_ref,
                y_s, res_s,
                s1_s, s2_s, s3_s):
    p = pl.program_id(0)
    i = pl.program_id(1)
    rows = pl.ds(i * _BLK, _BLK)
    ones_row = jnp.ones((1, _BLK), jnp.float32)

    @pl.when(p == 0)
    def _phase1():
        s0 = jnp.sum(st0_ref[...], axis=0)
        au, cu = _affine(s0[0:1], s0[1:2], gu_ref[...], bu_ref[...])
        am, cm = _affine(s0[2:3], s0[3:4], gm_ref[...], bm_ref[...])
        comb = jnp.concatenate([ue_ref[...] * au + cu,
                                me_ref[...] * am + cm], axis=1)
        y1 = jnp.dot(comb.astype(jnp.bfloat16), w1t_ref[...],
                     preferred_element_type=jnp.float32) + b1_ref[...]
        y_s[rows, :] = y1
        res_s[rows] = (jnp.sum(comb * wr_ref[...], axis=1)
                       + sc_ref[0, 0] + ub_ref[...] + mb_ref[...])

        _acc(s1_s, i, _colstats(y1, ones_row))

    @pl.when(p == 1)
    def _phase2():
        s1 = s1_s[...]
        a, c = _affine(s1[0:1], s1[1:2], g1_ref[...], be1_ref[...])
        x = jnp.maximum(y_s[rows, :] * a + c, 0.0)
        y2 = jnp.dot(x.astype(jnp.bfloat16), w2t_ref[...],
                     preferred_element_type=jnp.float32) + b2_ref[...]
        y_s[rows, 0:256] = y2
        _acc(s2_s, i, _colstats(y2, ones_row))

    @pl.when(p == 2)
    def _phase3():
        s2 = s2_s[...]
        a, c = _affine(s2[0:1], s2[1:2], g2_ref[...], be2_ref[...])
        x = jnp.maximum(y_s[rows, 0:256] * a + c, 0.0)
        y3 = jnp.dot(x.astype(jnp.bfloat16), w3t_ref[...],
                     preferred_element_type=jnp.float32) + b3_ref[...]
        y_s[rows, 0:128] = y3
        _acc(s3_s, i, _colstats(y3, ones_row))

    @pl.when(p == 3)
    def _phase4():
        s3 = s3_s[...]
        a, c = _affine(s3[0:1], s3[1:2], g3_ref[...], be3_ref[...])
        x = jnp.maximum(y_s[rows, 0:128] * a + c, 0.0)
        out_ref[...] = (jnp.sum(x * wf_ref[...], axis=1)
                        + bf_ref[0, 0] + res_s[rows])


def _emb_spec():
    return pl.BlockSpec((_BLK, _D), lambda p, i: (jnp.where(p == 0, i, 0), 0))


def _bias_spec():
    return pl.BlockSpec((_BLK,), lambda p, i: (jnp.where(p == 0, i, 0),))


def _full_spec(shape):
    nd = len(shape)
    return pl.BlockSpec(shape, lambda p, i: (0,) * nd)


def kernel(users, movies, user_emb, movie_emb, user_bias_t, movie_bias_t,
           global_bias, gu, bu, gm, bm, W1, b1, g1, be1, W2, b2, g2, be2,
           W3, b3, g3, be3, Wf, bf, Wr, br):
    uidx = users.astype(jnp.int32).reshape(_NW, _CH, 128)
    midx = movies.astype(jnp.int32).reshape(_NW, _CH, 128)
    ue, me, ubg, mbg, st0 = _sc_gather(uidx, midx, user_emb, movie_emb,
                                       user_bias_t.reshape(-1),
                                       movie_bias_t.reshape(-1))

    f32 = jnp.float32
    scalar_c = (global_bias + br).reshape(1, 1)
    out = pl.pallas_call(
        _fused_body,
        grid=(4, _NBLK),
        in_specs=[_emb_spec(), _emb_spec(),
                  _full_spec((_NW, 4, _D)),
                  _full_spec((1, _D)), _full_spec((1, _D)),
                  _full_spec((1, _D)), _full_spec((1, _D)),
                  _full_spec((2 * _D, 512)), _full_spec((1, 512)),
                  _full_spec((1, 2 * _D)), _full_spec((1, 1)),
                  _bias_spec(), _bias_spec(),
                  _full_spec((1, 512)), _full_spec((1, 512)),
                  _full_spec((512, 256)), _full_spec((1, 256)),
                  _full_spec((1, 256)), _full_spec((1, 256)),
                  _full_spec((256, _D)), _full_spec((1, _D)),
                  _full_spec((1, _D)), _full_spec((1, _D)),
                  _full_spec((1, _D)), _full_spec((1, 1))],
        out_specs=pl.BlockSpec((_BLK,),
                               lambda p, i: (jnp.where(p == 3, i, 0),)),
        out_shape=jax.ShapeDtypeStruct((_B,), f32),
        scratch_shapes=[
            pltpu.VMEM((_B, 512), f32),
            pltpu.VMEM((_B,), f32),
            pltpu.VMEM((2, 512), f32),
            pltpu.VMEM((2, 256), f32),
            pltpu.VMEM((2, _D), f32),
        ],
        compiler_params=pltpu.CompilerParams(
            vmem_limit_bytes=110 * 1024 * 1024,
        ),
    )(ue, me, st0, gu.reshape(1, -1), bu.reshape(1, -1),
      gm.reshape(1, -1), bm.reshape(1, -1), W1.T.astype(jnp.bfloat16), b1.reshape(1, -1),
      Wr, scalar_c, ubg, mbg,
      g1.reshape(1, -1), be1.reshape(1, -1), W2.T.astype(jnp.bfloat16), b2.reshape(1, -1),
      g2.reshape(1, -1), be2.reshape(1, -1), W3.T.astype(jnp.bfloat16), b3.reshape(1, -1),
      g3.reshape(1, -1), be3.reshape(1, -1), Wf, bf.reshape(1, 1))

    return out


# speedup vs baseline: 1.4141x; 1.0001x over previous
import functools

import jax
import jax.numpy as jnp
from jax import lax
from jax.experimental import pallas as pl
from jax.experimental.pallas import tpu as pltpu
from jax.experimental.pallas import tpu_sc as plsc

_B = 16384
_D = 128
_NC = 2
_NS = 16
_NW = _NC * _NS
_BPW = _B // _NW
_CH = _BPW // 128
_EPS = 1e-5

_BLK = 4096
_NBLK = _B // _BLK


def _chunk_stats(rows, j, sums, sqs):
    def body(r, carry):
        out = []
        for c in range(_D // 16):
            v = rows[j * 128 + r, pl.ds(c * 16, 16)]
            out.append(carry[c] + v)
            out.append(carry[_D // 16 + c] + v * v)
        return tuple(out[::2]) + tuple(out[1::2])

    carry = lax.fori_loop(0, 128, body, tuple(sums) + tuple(sqs))
    return list(carry[:_D // 16]), list(carry[_D // 16:])


def _sc_gather_body(uidx_hbm, midx_hbm, uemb_hbm, memb_hbm, ubias_hbm,
                    mbias_hbm, ue_out, me_out, ub_out, mb_out, st_out,
                    idx_u, idx_m, rows, brows_u, brows_m, st_v,
                    sem_g, sem_w, sem_b):
    wid = lax.axis_index("s") * _NC + lax.axis_index("c")
    base = wid * _BPW
    pltpu.sync_copy(uidx_hbm.at[wid], idx_u)
    pltpu.sync_copy(midx_hbm.at[wid], idx_m)
    hb = []
    for j in range(_CH):
        hb.append(pltpu.async_copy(ubias_hbm.at[idx_u.at[j]],
                                   brows_u.at[pl.ds(j * 128, 128)], sem_b))
        hb.append(pltpu.async_copy(mbias_hbm.at[idx_m.at[j]],
                                   brows_m.at[pl.ds(j * 128, 128)], sem_b))
    zeros = [jnp.zeros((16,), jnp.float32)] * (_D // 16)
    hg = [pltpu.async_copy(uemb_hbm.at[idx_u.at[j]],
                           rows.at[pl.ds(j * 128, 128)], sem_g.at[j])
          for j in range(_CH)]
    hw = []
    su, qu = zeros, zeros
    for j in range(_CH):
        hg[j].wait()
        hw.append(pltpu.async_copy(
            rows.at[pl.ds(j * 128, 128)],
            ue_out.at[pl.ds(base + j * 128, 128)], sem_w.at[j]))
        su, qu = _chunk_stats(rows, j, su, qu)
        hw[j].wait()
        hg.append(pltpu.async_copy(memb_hbm.at[idx_m.at[j]],
                                   rows.at[pl.ds(j * 128, 128)], sem_g.at[j]))
    sm, qm = zeros, zeros
    for j in range(_CH):
        hg[_CH + j].wait()
        hw.append(pltpu.async_copy(
            rows.at[pl.ds(j * 128, 128)],
            me_out.at[pl.ds(base + j * 128, 128)], sem_w.at[j]))
        sm, qm = _chunk_stats(rows, j, sm, qm)
    for c in range(_D // 16):
        st_v[0, pl.ds(c * 16, 16)] = su[c]
        st_v[1, pl.ds(c * 16, 16)] = qu[c]
        st_v[2, pl.ds(c * 16, 16)] = sm[c]
        st_v[3, pl.ds(c * 16, 16)] = qm[c]
    pltpu.sync_copy(st_v, st_out.at[wid])
    for h in hb:
        h.wait()
    pltpu.sync_copy(brows_u, ub_out.at[pl.ds(base, _BPW)])
    pltpu.sync_copy(brows_m, mb_out.at[pl.ds(base, _BPW)])
    for j in range(_CH):
        hw[_CH + j].wait()


def _sc_gather(uidx, midx, uemb, memb, ubias, mbias):
    mesh = plsc.VectorSubcoreMesh(core_axis_name="c", subcore_axis_name="s")
    fn = pl.kernel(
        _sc_gather_body,
        mesh=mesh,
        out_type=(
            jax.ShapeDtypeStruct((_B, _D), jnp.float32),
            jax.ShapeDtypeStruct((_B, _D), jnp.float32),
            jax.ShapeDtypeStruct((_B,), jnp.float32),
            jax.ShapeDtypeStruct((_B,), jnp.float32),
            jax.ShapeDtypeStruct((_NW, 4, _D), jnp.float32),
        ),
        scratch_types=[
            pltpu.VMEM((_CH, 128), jnp.int32),
            pltpu.VMEM((_CH, 128), jnp.int32),
            pltpu.VMEM((_BPW, _D), jnp.float32),
            pltpu.VMEM((_BPW,), jnp.float32),
            pltpu.VMEM((_BPW,), jnp.float32),
            pltpu.VMEM((4, _D), jnp.float32),
            pltpu.SemaphoreType.DMA((_CH,)),
            pltpu.SemaphoreType.DMA((_CH,)),
            pltpu.SemaphoreType.DMA,
        ],
    )
    return fn(uidx, midx, uemb, memb, ubias, mbias)


def _affine(s_sum, s_sq, g, be):
    mu = s_sum * (1.0 / _B)
    var = s_sq * (1.0 / _B) - mu * mu
    a = g * lax.rsqrt(var + _EPS)
    return a, be - mu * a


def _colstats(y, ones_row):
    s = jnp.dot(ones_row, y, preferred_element_type=jnp.float32)
    sq = jnp.dot(ones_row, y * y, preferred_element_type=jnp.float32)
    return jnp.concatenate([s, sq], axis=0)


def _acc(ref, i, val):
    @pl.when(i == 0)
    def _():
        ref[...] = val

    @pl.when(i > 0)
    def _():
        ref[...] += val


def _dott(x, w):
    return lax.dot_general(x, w, dimension_numbers=(((1,), (1,)), ((), ())),
                           preferred_element_type=jnp.float32)


def _fused_body(ue_ref, me_ref, st0_ref, gu_ref, bu_ref, gm_ref, bm_ref,
                w1_ref, b1_ref, wr_ref, gb_ref, br_ref, ub_ref, mb_ref,
                g1_ref, be1_ref, w2_ref, b2_ref,
                g2_ref, be2_ref, w3_ref, b3_ref,
                g3_ref, be3_ref, wf_ref, bf_ref,
                out_ref,
                y_s, res_s,
                s1_s, s2_s, s3_s):
    p = pl.program_id(0)
    i = pl.program_id(1)
    rows = pl.ds(i * _BLK, _BLK)
    ones_row = jnp.ones((1, _BLK), jnp.float32)

    @pl.when(p == 0)
    def _phase1():
        s0 = jnp.sum(st0_ref[...], axis=0)
        au, cu = _affine(s0[0], s0[1], gu_ref[...], bu_ref[...])
        am, cm = _affine(s0[2], s0[3], gm_ref[...], bm_ref[...])
        comb = jnp.concatenate([ue_ref[...] * au + cu,
                                me_ref[...] * am + cm], axis=1)
        y1 = _dott(comb, w1_ref[...]) + b1_ref[...]
        y_s[rows, :] = y1
        res_s[rows] = (jnp.sum(comb * wr_ref[...], axis=1)
                       + gb_ref[0] + br_ref[0] + ub_ref[...] + mb_ref[...])
        _acc(s1_s, i, _colstats(y1, ones_row))

    @pl.when(p == 1)
    def _phase2():
        s1 = s1_s[...]
        a, c = _affine(s1[0], s1[1], g1_ref[...], be1_ref[...])
        x = jnp.maximum(y_s[rows, :] * a + c, 0.0)
        y_s[rows, 0:256] = _dott(x, w2_ref[...]) + b2_ref[...]
        _acc(s2_s, i, _colstats(y_s[rows, 0:256], ones_row))

    @pl.when(p == 2)
    def _phase3():
        s2 = s2_s[...]
        a, c = _affine(s2[0], s2[1], g2_ref[...], be2_ref[...])
        x = jnp.maximum(y_s[rows, 0:256] * a + c, 0.0)
        y_s[rows, 0:128] = _dott(x, w3_ref[...]) + b3_ref[...]
        _acc(s3_s, i, _colstats(y_s[rows, 0:128], ones_row))

    @pl.when(p == 3)
    def _phase4():
        s3 = s3_s[...]
        a, c = _affine(s3[0], s3[1], g3_ref[...], be3_ref[...])
        x = jnp.maximum(y_s[rows, 0:128] * a + c, 0.0)
        out_ref[...] = (jnp.sum(x * wf_ref[...], axis=1)
                        + bf_ref[0] + res_s[rows])


def _emb_spec():
    return pl.BlockSpec((_BLK, _D), lambda p, i: (jnp.where(p == 0, i, 0), 0))


def _bias_spec():
    return pl.BlockSpec((_BLK,), lambda p, i: (jnp.where(p == 0, i, 0),))


def _full_spec(shape):
    nd = len(shape)
    return pl.BlockSpec(shape, lambda p, i: (0,) * nd)


def kernel(users, movies, user_emb, movie_emb, user_bias_t, movie_bias_t,
           global_bias, gu, bu, gm, bm, W1, b1, g1, be1, W2, b2, g2, be2,
           W3, b3, g3, be3, Wf, bf, Wr, br):
    uidx = users.astype(jnp.int32).reshape(_NW, _CH, 128)
    midx = movies.astype(jnp.int32).reshape(_NW, _CH, 128)
    ue, me, ubg, mbg, st0 = _sc_gather(uidx, midx, user_emb, movie_emb,
                                       user_bias_t.reshape(-1),
                                       movie_bias_t.reshape(-1))

    f32 = jnp.float32
    out = pl.pallas_call(
        _fused_body,
        grid=(4, _NBLK),
        in_specs=[_emb_spec(), _emb_spec(),
                  _full_spec((_NW, 4, _D)),
                  _full_spec((_D,)), _full_spec((_D,)),
                  _full_spec((_D,)), _full_spec((_D,)),
                  _full_spec((512, 2 * _D)), _full_spec((512,)),
                  _full_spec((1, 2 * _D)), _full_spec((1,)), _full_spec((1,)),
                  _bias_spec(), _bias_spec(),
                  _full_spec((512,)), _full_spec((512,)),
                  _full_spec((256, 512)), _full_spec((256,)),
                  _full_spec((256,)), _full_spec((256,)),
                  _full_spec((_D, 256)), _full_spec((_D,)),
                  _full_spec((_D,)), _full_spec((_D,)),
                  _full_spec((1, _D)), _full_spec((1,))],
        out_specs=pl.BlockSpec((_BLK,),
                               lambda p, i: (jnp.where(p == 3, i, 0),)),
        out_shape=jax.ShapeDtypeStruct((_B,), f32),
        scratch_shapes=[
            pltpu.VMEM((_B, 512), f32),
            pltpu.VMEM((_B,), f32),
            pltpu.VMEM((2, 512), f32),
            pltpu.VMEM((2, 256), f32),
            pltpu.VMEM((2, _D), f32),
        ],
        compiler_params=pltpu.CompilerParams(
            vmem_limit_bytes=110 * 1024 * 1024,
        ),
    )(ue, me, st0, gu, bu, gm, bm, W1, b1, Wr, global_bias, br, ubg, mbg,
      g1, be1, W2, b2, g2, be2, W3, b3, g3, be3, Wf, bf)

    return out


# speedup vs baseline: 1.4328x; 1.0132x over previous
import functools

import jax
import jax.numpy as jnp
from jax import lax
from jax.experimental import pallas as pl
from jax.experimental.pallas import tpu as pltpu
from jax.experimental.pallas import tpu_sc as plsc

_B = 16384
_D = 128
_NC = 2
_NS = 16
_NW = _NC * _NS
_BPW = _B // _NW
_CH = _BPW // 128
_EPS = 1e-5

_BLK = 4096
_NBLK = _B // _BLK


def _chunk_stats(rows, j, sums, sqs):
    def body(r, carry):
        out = []
        for c in range(_D // 16):
            v = rows[j * 128 + r, pl.ds(c * 16, 16)]
            out.append(carry[c] + v)
            out.append(carry[_D // 16 + c] + v * v)
        return tuple(out[::2]) + tuple(out[1::2])

    carry = lax.fori_loop(0, 128, body, tuple(sums) + tuple(sqs))
    return list(carry[:_D // 16]), list(carry[_D // 16:])


def _sc_gather_body(uidx_hbm, midx_hbm, uemb_hbm, memb_hbm, ubias_hbm,
                    mbias_hbm, ue_out, me_out, ub_out, mb_out, st_out,
                    idx_u, idx_m, rows, brows_u, brows_m, st_v,
                    sem_g, sem_w, sem_b):
    wid = lax.axis_index("s") * _NC + lax.axis_index("c")
    base = wid * _BPW
    pltpu.sync_copy(uidx_hbm.at[wid], idx_u)
    pltpu.sync_copy(midx_hbm.at[wid], idx_m)
    hb = []
    for j in range(_CH):
        hb.append(pltpu.async_copy(ubias_hbm.at[idx_u.at[j]],
                                   brows_u.at[pl.ds(j * 128, 128)], sem_b))
        hb.append(pltpu.async_copy(mbias_hbm.at[idx_m.at[j]],
                                   brows_m.at[pl.ds(j * 128, 128)], sem_b))
    zeros = [jnp.zeros((16,), jnp.float32)] * (_D // 16)
    hg = [pltpu.async_copy(uemb_hbm.at[idx_u.at[j]],
                           rows.at[pl.ds(j * 128, 128)], sem_g.at[j])
          for j in range(_CH)]
    hw = []
    su, qu = zeros, zeros
    for j in range(_CH):
        hg[j].wait()
        hw.append(pltpu.async_copy(
            rows.at[pl.ds(j * 128, 128)],
            ue_out.at[pl.ds(base + j * 128, 128)], sem_w.at[j]))
        su, qu = _chunk_stats(rows, j, su, qu)
        hw[j].wait()
        hg.append(pltpu.async_copy(memb_hbm.at[idx_m.at[j]],
                                   rows.at[pl.ds(j * 128, 128)], sem_g.at[j]))
    sm, qm = zeros, zeros
    for j in range(_CH):
        hg[_CH + j].wait()
        hw.append(pltpu.async_copy(
            rows.at[pl.ds(j * 128, 128)],
            me_out.at[pl.ds(base + j * 128, 128)], sem_w.at[j]))
        sm, qm = _chunk_stats(rows, j, sm, qm)
    for c in range(_D // 16):
        st_v[0, pl.ds(c * 16, 16)] = su[c]
        st_v[1, pl.ds(c * 16, 16)] = qu[c]
        st_v[2, pl.ds(c * 16, 16)] = sm[c]
        st_v[3, pl.ds(c * 16, 16)] = qm[c]
    pltpu.sync_copy(st_v, st_out.at[wid])
    for h in hb:
        h.wait()
    pltpu.sync_copy(brows_u, ub_out.at[pl.ds(base, _BPW)])
    pltpu.sync_copy(brows_m, mb_out.at[pl.ds(base, _BPW)])
    for j in range(_CH):
        hw[_CH + j].wait()


def _sc_gather(uidx, midx, uemb, memb, ubias, mbias):
    mesh = plsc.VectorSubcoreMesh(core_axis_name="c", subcore_axis_name="s")
    fn = pl.kernel(
        _sc_gather_body,
        mesh=mesh,
        out_type=(
            jax.ShapeDtypeStruct((_B, _D), jnp.float32),
            jax.ShapeDtypeStruct((_B, _D), jnp.float32),
            jax.ShapeDtypeStruct((_B,), jnp.float32),
            jax.ShapeDtypeStruct((_B,), jnp.float32),
            jax.ShapeDtypeStruct((_NW, 4, _D), jnp.float32),
        ),
        scratch_types=[
            pltpu.VMEM((_CH, 128), jnp.int32),
            pltpu.VMEM((_CH, 128), jnp.int32),
            pltpu.VMEM((_BPW, _D), jnp.float32),
            pltpu.VMEM((_BPW,), jnp.float32),
            pltpu.VMEM((_BPW,), jnp.float32),
            pltpu.VMEM((4, _D), jnp.float32),
            pltpu.SemaphoreType.DMA((_CH,)),
            pltpu.SemaphoreType.DMA((_CH,)),
            pltpu.SemaphoreType.DMA,
        ],
    )
    return fn(uidx, midx, uemb, memb, ubias, mbias)


def _affine(s_sum, s_sq, g, be):
    mu = s_sum * (1.0 / _B)
    var = s_sq * (1.0 / _B) - mu * mu
    a = g * lax.rsqrt(var + _EPS)
    return a, be - mu * a


def _colstats(y, ones_row):
    s = jnp.dot(ones_row, y, preferred_element_type=jnp.float32)
    sq = jnp.dot(ones_row, y * y, preferred_element_type=jnp.float32)
    return jnp.concatenate([s, sq], axis=0)


def _acc(ref, i, val):
    @pl.when(i == 0)
    def _():
        ref[...] = val

    @pl.when(i > 0)
    def _():
        ref[...] += val


def _dott(x, w):
    return lax.dot_general(x, w, dimension_numbers=(((1,), (1,)), ((), ())),
                           preferred_element_type=jnp.float32)


def _fused_body(ue_ref, me_ref, st0_ref, gu_ref, bu_ref, gm_ref, bm_ref,
                w1_ref, b1_ref, wr_ref, gb_ref, br_ref, ub_ref, mb_ref,
                g1_ref, be1_ref, w2_ref, b2_ref,
                g2_ref, be2_ref, w3_ref, b3_ref,
                g3_ref, be3_ref, wf_ref, bf_ref,
                out_ref,
                y_s, res_s,
                s1_s, s2_s, s3_s):
    p = pl.program_id(0)
    i = pl.program_id(1)
    rows = pl.ds(i * _BLK, _BLK)
    ones_row = jnp.ones((1, _BLK), jnp.float32)

    @pl.when(p == 0)
    def _phase1():
        s0 = jnp.sum(st0_ref[...], axis=0)
        au, cu = _affine(s0[0], s0[1], gu_ref[...], bu_ref[...])
        am, cm = _affine(s0[2], s0[3], gm_ref[...], bm_ref[...])
        comb = jnp.concatenate([ue_ref[...] * au + cu,
                                me_ref[...] * am + cm], axis=1)
        y1 = _dott(comb, w1_ref[...]) + b1_ref[...]
        y_s[rows, :] = y1
        res_s[rows] = (jnp.sum(comb * wr_ref[...], axis=1)
                       + gb_ref[0] + br_ref[0] + ub_ref[...] + mb_ref[...])
        sq = jnp.dot(ones_row, y1 * y1, preferred_element_type=jnp.float32)

        @pl.when(i == 0)
        def _():
            cs = jnp.concatenate([au * s0[0] + _B * cu,
                                  am * s0[2] + _B * cm])
            s1_s[0, :] = _dott(cs[None, :], w1_ref[...])[0] + _B * b1_ref[...]
            s1_s[1, :] = sq[0]

        @pl.when(i > 0)
        def _():
            s1_s[1, :] += sq[0]

    @pl.when(p == 1)
    def _phase2():
        s1 = s1_s[...]
        a, c = _affine(s1[0], s1[1], g1_ref[...], be1_ref[...])
        x = jnp.maximum(y_s[rows, :] * a + c, 0.0)
        y_s[rows, 0:256] = _dott(x, w2_ref[...]) + b2_ref[...]
        _acc(s2_s, i, _colstats(y_s[rows, 0:256], ones_row))

    @pl.when(p == 2)
    def _phase3():
        s2 = s2_s[...]
        a, c = _affine(s2[0], s2[1], g2_ref[...], be2_ref[...])
        x = jnp.maximum(y_s[rows, 0:256] * a + c, 0.0)
        y_s[rows, 0:128] = _dott(x, w3_ref[...]) + b3_ref[...]
        _acc(s3_s, i, _colstats(y_s[rows, 0:128], ones_row))

    @pl.when(p == 3)
    def _phase4():
        s3 = s3_s[...]
        a, c = _affine(s3[0], s3[1], g3_ref[...], be3_ref[...])
        x = jnp.maximum(y_s[rows, 0:128] * a + c, 0.0)
        out_ref[...] = (jnp.sum(x * wf_ref[...], axis=1)
                        + bf_ref[0] + res_s[rows])


def _emb_spec():
    return pl.BlockSpec((_BLK, _D), lambda p, i: (jnp.where(p == 0, i, 0), 0))


def _bias_spec():
    return pl.BlockSpec((_BLK,), lambda p, i: (jnp.where(p == 0, i, 0),))


def _full_spec(shape):
    nd = len(shape)
    return pl.BlockSpec(shape, lambda p, i: (0,) * nd)


def kernel(users, movies, user_emb, movie_emb, user_bias_t, movie_bias_t,
           global_bias, gu, bu, gm, bm, W1, b1, g1, be1, W2, b2, g2, be2,
           W3, b3, g3, be3, Wf, bf, Wr, br):
    uidx = users.astype(jnp.int32).reshape(_NW, _CH, 128)
    midx = movies.astype(jnp.int32).reshape(_NW, _CH, 128)
    ue, me, ubg, mbg, st0 = _sc_gather(uidx, midx, user_emb, movie_emb,
                                       user_bias_t.reshape(-1),
                                       movie_bias_t.reshape(-1))

    f32 = jnp.float32
    out = pl.pallas_call(
        _fused_body,
        grid=(4, _NBLK),
        in_specs=[_emb_spec(), _emb_spec(),
                  _full_spec((_NW, 4, _D)),
                  _full_spec((_D,)), _full_spec((_D,)),
                  _full_spec((_D,)), _full_spec((_D,)),
                  _full_spec((512, 2 * _D)), _full_spec((512,)),
                  _full_spec((1, 2 * _D)), _full_spec((1,)), _full_spec((1,)),
                  _bias_spec(), _bias_spec(),
                  _full_spec((512,)), _full_spec((512,)),
                  _full_spec((256, 512)), _full_spec((256,)),
                  _full_spec((256,)), _full_spec((256,)),
                  _full_spec((_D, 256)), _full_spec((_D,)),
                  _full_spec((_D,)), _full_spec((_D,)),
                  _full_spec((1, _D)), _full_spec((1,))],
        out_specs=pl.BlockSpec((_BLK,),
                               lambda p, i: (jnp.where(p == 3, i, 0),)),
        out_shape=jax.ShapeDtypeStruct((_B,), f32),
        scratch_shapes=[
            pltpu.VMEM((_B, 512), f32),
            pltpu.VMEM((_B,), f32),
            pltpu.VMEM((2, 512), f32),
            pltpu.VMEM((2, 256), f32),
            pltpu.VMEM((2, _D), f32),
        ],
        compiler_params=pltpu.CompilerParams(
            vmem_limit_bytes=110 * 1024 * 1024,
        ),
    )(ue, me, st0, gu, bu, gm, bm, W1, b1, Wr, global_bias, br, ubg, mbg,
      g1, be1, W2, b2, g2, be2, W3, b3, g3, be3, Wf, bf)

    return out


# speedup vs baseline: 1.4437x; 1.0076x over previous
import jax
import jax.numpy as jnp
from jax import lax
from jax.experimental import pallas as pl
from jax.experimental.pallas import tpu as pltpu
from jax.experimental.pallas import tpu_sc as plsc

_B = 16384
_D = 128
_NC = 2
_NS = 16
_NW = _NC * _NS
_BPW = _B // _NW
_CH = _BPW // 128
_EPS = 1e-5

_BLK = 4096
_NBLK = _B // _BLK


def _chunk_stats(rows, j, sums, sqs):
    def body(r, carry):
        out = []
        for c in range(_D // 16):
            v = rows[j * 128 + r, pl.ds(c * 16, 16)]
            out.append(carry[c] + v)
            out.append(carry[_D // 16 + c] + v * v)
        return tuple(out[::2]) + tuple(out[1::2])

    carry = lax.fori_loop(0, 128, body, tuple(sums) + tuple(sqs))
    return list(carry[:_D // 16]), list(carry[_D // 16:])


def _sc_gather_body(uidx_hbm, midx_hbm, uemb_hbm, memb_hbm, ubias_hbm,
                    mbias_hbm, ue_out, me_out, ub_out, mb_out, st_out,
                    idx_u, idx_m, rows, brows_u, brows_m, st_v,
                    sem_g, sem_w, sem_b):
    wid = lax.axis_index("s") * _NC + lax.axis_index("c")
    base = wid * _BPW
    pltpu.sync_copy(uidx_hbm.at[wid], idx_u)
    pltpu.sync_copy(midx_hbm.at[wid], idx_m)
    hb = []
    for j in range(_CH):
        hb.append(pltpu.async_copy(ubias_hbm.at[idx_u.at[j]],
                                   brows_u.at[pl.ds(j * 128, 128)], sem_b))
        hb.append(pltpu.async_copy(mbias_hbm.at[idx_m.at[j]],
                                   brows_m.at[pl.ds(j * 128, 128)], sem_b))
    zeros = [jnp.zeros((16,), jnp.float32)] * (_D // 16)
    hg = [pltpu.async_copy(uemb_hbm.at[idx_u.at[j]],
                           rows.at[pl.ds(j * 128, 128)], sem_g.at[j])
          for j in range(_CH)]
    hw = []
    su, qu = zeros, zeros
    for j in range(_CH):
        hg[j].wait()
        hw.append(pltpu.async_copy(
            rows.at[pl.ds(j * 128, 128)],
            ue_out.at[pl.ds(base + j * 128, 128)], sem_w.at[j]))
        su, qu = _chunk_stats(rows, j, su, qu)
        hw[j].wait()
        hg.append(pltpu.async_copy(memb_hbm.at[idx_m.at[j]],
                                   rows.at[pl.ds(j * 128, 128)], sem_g.at[j]))
    sm, qm = zeros, zeros
    for j in range(_CH):
        hg[_CH + j].wait()
        hw.append(pltpu.async_copy(
            rows.at[pl.ds(j * 128, 128)],
            me_out.at[pl.ds(base + j * 128, 128)], sem_w.at[j]))
        sm, qm = _chunk_stats(rows, j, sm, qm)
    for c in range(_D // 16):
        st_v[0, pl.ds(c * 16, 16)] = su[c]
        st_v[1, pl.ds(c * 16, 16)] = qu[c]
        st_v[2, pl.ds(c * 16, 16)] = sm[c]
        st_v[3, pl.ds(c * 16, 16)] = qm[c]
    pltpu.sync_copy(st_v, st_out.at[wid])
    for h in hb:
        h.wait()
    pltpu.sync_copy(brows_u, ub_out.at[pl.ds(base, _BPW)])
    pltpu.sync_copy(brows_m, mb_out.at[pl.ds(base, _BPW)])
    for j in range(_CH):
        hw[_CH + j].wait()


def _sc_gather(uidx, midx, uemb, memb, ubias, mbias):
    mesh = plsc.VectorSubcoreMesh(core_axis_name="c", subcore_axis_name="s")
    fn = pl.kernel(
        _sc_gather_body,
        mesh=mesh,
        out_type=(
            jax.ShapeDtypeStruct((_B, _D), jnp.float32),
            jax.ShapeDtypeStruct((_B, _D), jnp.float32),
            jax.ShapeDtypeStruct((_B,), jnp.float32),
            jax.ShapeDtypeStruct((_B,), jnp.float32),
            jax.ShapeDtypeStruct((_NW, 4, _D), jnp.float32),
        ),
        scratch_types=[
            pltpu.VMEM((_CH, 128), jnp.int32),
            pltpu.VMEM((_CH, 128), jnp.int32),
            pltpu.VMEM((_BPW, _D), jnp.float32),
            pltpu.VMEM((_BPW,), jnp.float32),
            pltpu.VMEM((_BPW,), jnp.float32),
            pltpu.VMEM((4, _D), jnp.float32),
            pltpu.SemaphoreType.DMA((_CH,)),
            pltpu.SemaphoreType.DMA((_CH,)),
            pltpu.SemaphoreType.DMA,
        ],
    )
    return fn(uidx, midx, uemb, memb, ubias, mbias)


def _affine(s_sum, s_sq, g, be):
    mu = s_sum * (1.0 / _B)
    var = s_sq * (1.0 / _B) - mu * mu
    a = g * lax.rsqrt(var + _EPS)
    return a, be - mu * a


def _colstats(y, ones_row):
    s = jnp.dot(ones_row, y, preferred_element_type=jnp.float32)
    sq = jnp.dot(ones_row, y * y, preferred_element_type=jnp.float32)
    return jnp.concatenate([s, sq], axis=0)


def _acc(ref, i, val):
    @pl.when(i == 0)
    def _():
        ref[...] = val

    @pl.when(i > 0)
    def _():
        ref[...] += val


def _dott(x, w):
    return lax.dot_general(x, w, dimension_numbers=(((1,), (1,)), ((), ())),
                           preferred_element_type=jnp.float32)


def _fused_body(ue_ref, me_ref, st0_ref, gu_ref, bu_ref, gm_ref, bm_ref,
                w1_ref, b1_ref, wr_ref, gb_ref, br_ref, ub_ref, mb_ref,
                g1_ref, be1_ref, w2_ref, b2_ref,
                g2_ref, be2_ref, w3_ref, b3_ref,
                g3_ref, be3_ref, wf_ref, bf_ref,
                out_ref,
                y_s, res_s,
                s1_s, s2_s, s3_s):
    p = pl.program_id(0)
    i = pl.program_id(1)
    rows = pl.ds(i * _BLK, _BLK)
    ones_row = jnp.ones((1, _BLK), jnp.float32)

    @pl.when(p == 0)
    def _phase1():
        s0 = jnp.sum(st0_ref[...], axis=0)
        au, cu = _affine(s0[0], s0[1], gu_ref[...], bu_ref[...])
        am, cm = _affine(s0[2], s0[3], gm_ref[...], bm_ref[...])
        comb = jnp.concatenate([ue_ref[...] * au + cu,
                                me_ref[...] * am + cm], axis=1)
        y1 = _dott(comb, w1_ref[...]) + b1_ref[...]
        y_s[rows, :] = y1
        res_s[rows] = (jnp.sum(comb * wr_ref[...], axis=1)
                       + gb_ref[0] + br_ref[0] + ub_ref[...] + mb_ref[...])
        sq = jnp.dot(ones_row, y1 * y1, preferred_element_type=jnp.float32)

        @pl.when(i == 0)
        def _():
            cs = jnp.concatenate([au * s0[0] + _B * cu,
                                  am * s0[2] + _B * cm])
            s1_s[0, :] = _dott(cs[None, :], w1_ref[...])[0] + _B * b1_ref[...]
            s1_s[1, :] = sq[0]

        @pl.when(i > 0)
        def _():
            s1_s[1, :] += sq[0]

    @pl.when(p == 1)
    def _phase2():
        s1 = s1_s[...]
        a, c = _affine(s1[0], s1[1], g1_ref[...], be1_ref[...])
        x = jnp.maximum(y_s[rows, :] * a + c, 0.0)
        y_s[rows, 0:256] = _dott(x, w2_ref[...]) + b2_ref[...]
        _acc(s2_s, i, _colstats(y_s[rows, 0:256], ones_row))

    @pl.when(p == 2)
    def _phase3():
        s2 = s2_s[...]
        a, c = _affine(s2[0], s2[1], g2_ref[...], be2_ref[...])
        x = jnp.maximum(y_s[rows, 0:256] * a + c, 0.0)
        y_s[rows, 0:128] = _dott(x, w3_ref[...]) + b3_ref[...]
        _acc(s3_s, i, _colstats(y_s[rows, 0:128], ones_row))

    @pl.when(p == 3)
    def _phase4():
        s3 = s3_s[...]
        a, c = _affine(s3[0], s3[1], g3_ref[...], be3_ref[...])
        x = jnp.maximum(y_s[rows, 0:128] * a + c, 0.0)
        out_ref[...] = (jnp.sum(x * wf_ref[...], axis=1)
                        + bf_ref[0] + res_s[rows])


def _emb_spec():
    return pl.BlockSpec((_BLK, _D), lambda p, i: (jnp.where(p == 0, i, 0), 0))


def _bias_spec():
    return pl.BlockSpec((_BLK,), lambda p, i: (jnp.where(p == 0, i, 0),))


def _full_spec(shape):
    nd = len(shape)
    return pl.BlockSpec(shape, lambda p, i: (0,) * nd)


def kernel(users, movies, user_emb, movie_emb, user_bias_t, movie_bias_t,
           global_bias, gu, bu, gm, bm, W1, b1, g1, be1, W2, b2, g2, be2,
           W3, b3, g3, be3, Wf, bf, Wr, br):
    uidx = users.astype(jnp.int32).reshape(_NW, _CH, 128)
    midx = movies.astype(jnp.int32).reshape(_NW, _CH, 128)
    ue, me, ubg, mbg, st0 = _sc_gather(uidx, midx, user_emb, movie_emb,
                                       user_bias_t.reshape(-1),
                                       movie_bias_t.reshape(-1))

    f32 = jnp.float32
    out = pl.pallas_call(
        _fused_body,
        grid=(4, _NBLK),
        in_specs=[_emb_spec(), _emb_spec(),
                  _full_spec((_NW, 4, _D)),
                  _full_spec((_D,)), _full_spec((_D,)),
                  _full_spec((_D,)), _full_spec((_D,)),
                  _full_spec((512, 2 * _D)), _full_spec((512,)),
                  _full_spec((1, 2 * _D)), _full_spec((1,)), _full_spec((1,)),
                  _bias_spec(), _bias_spec(),
                  _full_spec((512,)), _full_spec((512,)),
                  _full_spec((256, 512)), _full_spec((256,)),
                  _full_spec((256,)), _full_spec((256,)),
                  _full_spec((_D, 256)), _full_spec((_D,)),
                  _full_spec((_D,)), _full_spec((_D,)),
                  _full_spec((1, _D)), _full_spec((1,))],
        out_specs=pl.BlockSpec((_BLK,),
                               lambda p, i: (jnp.where(p == 3, i, 0),)),
        out_shape=jax.ShapeDtypeStruct((_B,), f32),
        scratch_shapes=[
            pltpu.VMEM((_B, 512), f32),
            pltpu.VMEM((_B,), f32),
            pltpu.VMEM((2, 512), f32),
            pltpu.VMEM((2, 256), f32),
            pltpu.VMEM((2, _D), f32),
        ],
        compiler_params=pltpu.CompilerParams(
            vmem_limit_bytes=110 * 1024 * 1024,
        ),
    )(ue, me, st0, gu, bu, gm, bm, W1, b1, Wr, global_bias, br, ubg, mbg,
      g1, be1, W2, b2, g2, be2, W3, b3, g3, be3, Wf, bf)

    return out
